# baseline (speedup 1.0000x reference)
import jax
import jax.numpy as jnp
from jax import lax
from jax.experimental import pallas as pl
from jax.experimental.pallas import tpu as pltpu

NUM_CODE = 8192
CODE_DIM = 32
TOK_TILE = 256
CHUNK = 2048
NCHUNK = NUM_CODE // CHUNK


def _vq_kernel(xb_ref, x2_ref, cb_ref, c2_ref, idx_ref, codes_ref):
    xb = xb_ref[...].astype(jnp.float32)
    x2 = x2_ref[...]

    best_v = jnp.full((TOK_TILE, 1), jnp.inf, jnp.bfloat16)
    best_i = jnp.zeros((TOK_TILE, 1), jnp.int32)
    for t in range(NCHUNK):
        c_t = cb_ref[t * CHUNK:(t + 1) * CHUNK, :]
        c2_t = c2_ref[:, t * CHUNK:(t + 1) * CHUNK]
        d = lax.dot_general(xb, c_t, (((1,), (1,)), ((), ())),
                            precision=lax.Precision.DEFAULT,
                            preferred_element_type=jnp.float32)
        v = (x2 - 2.0 * d) + c2_t
        m = jnp.min(v, axis=1, keepdims=True)
        iota = lax.broadcasted_iota(jnp.int32, v.shape, 1)
        ii = jnp.min(jnp.where(v == m, iota, NUM_CODE),
                     axis=1, keepdims=True) + t * CHUNK
        upd = m < best_v.astype(jnp.float32)
        best_i = jnp.where(upd, ii, best_i)
        best_v = jnp.where(upd, m.astype(jnp.bfloat16), best_v)

    idx_ref[...] = best_i
    oh_iota = lax.broadcasted_iota(jnp.int32, (TOK_TILE, NUM_CODE), 1)
    onehot = (oh_iota == best_i).astype(jnp.float32)
    cb = cb_ref[...]
    cb_hi = cb.astype(jnp.bfloat16).astype(jnp.float32)
    cb_lo = cb - cb_hi
    dims = (((1,), (0,)), ((), ()))
    codes_ref[...] = (
        lax.dot_general(onehot, cb_hi, dims,
                        precision=lax.Precision.DEFAULT,
                        preferred_element_type=jnp.float32)
        + lax.dot_general(onehot, cb_lo, dims,
                          precision=lax.Precision.DEFAULT,
                          preferred_element_type=jnp.float32))


@jax.jit
def kernel(z_e, codebook):
    B, C, H, W = z_e.shape
    N = B * H * W
    z = jnp.transpose(z_e, (0, 2, 3, 1))
    flat = z.reshape(-1, C)
    x2 = jnp.sum(flat ** 2, axis=1, keepdims=True)
    c2 = jnp.sum(codebook ** 2, axis=1)[None, :]
    xb = flat.astype(jnp.bfloat16)

    grid = (N // TOK_TILE,)
    idx2d, codes = pl.pallas_call(
        _vq_kernel,
        grid=grid,
        in_specs=[
            pl.BlockSpec((TOK_TILE, C), lambda i: (i, 0)),
            pl.BlockSpec((TOK_TILE, 1), lambda i: (i, 0)),
            pl.BlockSpec((NUM_CODE, C), lambda i: (0, 0)),
            pl.BlockSpec((1, NUM_CODE), lambda i: (0, 0)),
        ],
        out_specs=[
            pl.BlockSpec((TOK_TILE, 1), lambda i: (i, 0)),
            pl.BlockSpec((TOK_TILE, C), lambda i: (i, 0)),
        ],
        out_shape=[
            jax.ShapeDtypeStruct((N, 1), jnp.int32),
            jax.ShapeDtypeStruct((N, C), jnp.float32),
        ],
        compiler_params=pltpu.CompilerParams(
            dimension_semantics=("parallel",)),
    )(xb, x2, codebook, c2)

    indices = idx2d.reshape(B, H, W)
    z_q = jnp.transpose(codes.reshape(B, H, W, C), (0, 3, 1, 2))
    return (z_q, z_q, indices)


# speedup vs baseline: 1.6054x; 1.6054x over previous
import functools

import jax
import jax.numpy as jnp
from jax import lax
from jax.experimental import pallas as pl
from jax.experimental.pallas import tpu as pltpu
from jax.experimental.pallas import tpu_sc as plsc

NUM_CODE = 8192
CODE_DIM = 32
TOK_TILE = 256
CHUNK = 2048
NCHUNK = NUM_CODE // CHUNK


def _vq_idx_kernel(xb_ref, x2_ref, cb_ref, c2_ref, idx_ref):
    xb = xb_ref[...].astype(jnp.float32)
    x2 = x2_ref[...]

    best_v = jnp.full((TOK_TILE, 1), jnp.inf, jnp.bfloat16)
    best_i = jnp.zeros((TOK_TILE, 1), jnp.int32)
    for t in range(NCHUNK):
        c_t = cb_ref[t * CHUNK:(t + 1) * CHUNK, :]
        c2_t = c2_ref[:, t * CHUNK:(t + 1) * CHUNK]
        d = lax.dot_general(xb, c_t, (((1,), (1,)), ((), ())),
                            precision=lax.Precision.DEFAULT,
                            preferred_element_type=jnp.float32)
        v = (x2 - 2.0 * d) + c2_t
        m = jnp.min(v, axis=1, keepdims=True)
        iota = lax.broadcasted_iota(jnp.int32, v.shape, 1)
        ii = jnp.min(jnp.where(v == m, iota, NUM_CODE),
                     axis=1, keepdims=True) + t * CHUNK
        upd = m < best_v.astype(jnp.float32)
        best_i = jnp.where(upd, ii, best_i)
        best_v = jnp.where(upd, m.astype(jnp.bfloat16), best_v)

    idx_ref[...] = best_i


def _make_sc_gather(V, D, B):
    info = plsc.get_sparse_core_info()
    NW = info.num_cores * info.num_subcores
    b_per_w = B // NW
    mesh = plsc.VectorSubcoreMesh(core_axis_name="c", subcore_axis_name="s")

    @functools.partial(
        pl.kernel, mesh=mesh,
        out_type=jax.ShapeDtypeStruct((B, D), jnp.float32),
        scratch_types=[
            pltpu.VMEM((b_per_w,), jnp.int32),
            pltpu.VMEM((b_per_w, D), jnp.float32),
            pltpu.SemaphoreType.DMA,
        ],
    )
    def gather(table_hbm, idx_hbm, out_hbm, idx_v, rows_v, sem):
        wid = lax.axis_index("s") * info.num_cores + lax.axis_index("c")
        base = wid * b_per_w
        pltpu.sync_copy(idx_hbm.at[pl.ds(base, b_per_w)], idx_v)
        pltpu.async_copy(table_hbm.at[idx_v], rows_v, sem).wait()
        pltpu.sync_copy(rows_v, out_hbm.at[pl.ds(base, b_per_w)])

    return gather


@jax.jit
def kernel(z_e, codebook):
    B, C, H, W = z_e.shape
    N = B * H * W
    z = jnp.transpose(z_e, (0, 2, 3, 1))
    flat = z.reshape(-1, C)
    x2 = jnp.sum(flat ** 2, axis=1, keepdims=True)
    c2 = jnp.sum(codebook ** 2, axis=1)[None, :]
    xb = flat.astype(jnp.bfloat16)

    grid = (N // TOK_TILE,)
    idx2d = pl.pallas_call(
        _vq_idx_kernel,
        grid=grid,
        in_specs=[
            pl.BlockSpec((TOK_TILE, C), lambda i: (i, 0)),
            pl.BlockSpec((TOK_TILE, 1), lambda i: (i, 0)),
            pl.BlockSpec((NUM_CODE, C), lambda i: (0, 0)),
            pl.BlockSpec((1, NUM_CODE), lambda i: (0, 0)),
        ],
        out_specs=pl.BlockSpec((TOK_TILE, 1), lambda i: (i, 0)),
        out_shape=jax.ShapeDtypeStruct((N, 1), jnp.int32),
        compiler_params=pltpu.CompilerParams(
            dimension_semantics=("parallel",)),
    )(xb, x2, codebook, c2)

    indices_flat = idx2d.reshape(N)
    cb_pad = jnp.pad(codebook, ((0, 0), (0, 128 - C)))
    codes = _make_sc_gather(NUM_CODE, 128, N)(cb_pad, indices_flat)[:, :C]

    indices = indices_flat.reshape(B, H, W)
    z_q = jnp.transpose(codes.reshape(B, H, W, C), (0, 3, 1, 2))
    return (z_q, z_q, indices)


# speedup vs baseline: 1.6759x; 1.0439x over previous
import functools

import jax
import jax.numpy as jnp
from jax import lax
from jax.experimental import pallas as pl
from jax.experimental.pallas import tpu as pltpu
from jax.experimental.pallas import tpu_sc as plsc

NUM_CODE = 8192
CODE_DIM = 32
TOK_TILE = 512
CHUNK = 2048
NCHUNK = NUM_CODE // CHUNK


def _vq_idx_kernel(xb_ref, x2_ref, cb_ref, c2_ref, idx_ref):
    xb = xb_ref[...].astype(jnp.float32)
    x2 = x2_ref[...]

    best_v = jnp.full((TOK_TILE, 1), jnp.inf, jnp.bfloat16)
    best_i = jnp.zeros((TOK_TILE, 1), jnp.int32)
    for t in range(NCHUNK):
        c_t = cb_ref[t * CHUNK:(t + 1) * CHUNK, :]
        c2_t = c2_ref[:, t * CHUNK:(t + 1) * CHUNK]
        d = lax.dot_general(xb, c_t, (((1,), (1,)), ((), ())),
                            precision=lax.Precision.DEFAULT,
                            preferred_element_type=jnp.float32)
        v = (x2 - 2.0 * d) + c2_t
        m = jnp.min(v, axis=1, keepdims=True)
        iota = lax.broadcasted_iota(jnp.int32, v.shape, 1)
        ii = jnp.min(jnp.where(v == m, iota, NUM_CODE),
                     axis=1, keepdims=True) + t * CHUNK
        upd = m < best_v.astype(jnp.float32)
        best_i = jnp.where(upd, ii, best_i)
        best_v = jnp.where(upd, m.astype(jnp.bfloat16), best_v)

    idx_ref[...] = best_i


def _make_sc_gather(V, D, B):
    info = plsc.get_sparse_core_info()
    NW = info.num_cores * info.num_subcores
    b_per_w = B // NW
    mesh = plsc.VectorSubcoreMesh(core_axis_name="c", subcore_axis_name="s")

    @functools.partial(
        pl.kernel, mesh=mesh,
        out_type=jax.ShapeDtypeStruct((B, D), jnp.float32),
        scratch_types=[
            pltpu.VMEM((b_per_w,), jnp.int32),
            pltpu.VMEM((b_per_w, D), jnp.float32),
            pltpu.SemaphoreType.DMA,
        ],
    )
    def gather(table_hbm, idx_hbm, out_hbm, idx_v, rows_v, sem):
        wid = lax.axis_index("s") * info.num_cores + lax.axis_index("c")
        base = wid * b_per_w
        pltpu.sync_copy(idx_hbm.at[pl.ds(base, b_per_w)], idx_v)
        pltpu.async_copy(table_hbm.at[idx_v], rows_v, sem).wait()
        pltpu.sync_copy(rows_v, out_hbm.at[pl.ds(base, b_per_w)])

    return gather


@jax.jit
def kernel(z_e, codebook):
    B, C, H, W = z_e.shape
    N = B * H * W
    z = jnp.transpose(z_e, (0, 2, 3, 1))
    flat = z.reshape(-1, C)
    x2 = jnp.sum(flat ** 2, axis=1, keepdims=True)
    c2 = jnp.sum(codebook ** 2, axis=1)[None, :]
    xb = flat.astype(jnp.bfloat16)

    grid = (N // TOK_TILE,)
    idx2d = pl.pallas_call(
        _vq_idx_kernel,
        grid=grid,
        in_specs=[
            pl.BlockSpec((TOK_TILE, C), lambda i: (i, 0)),
            pl.BlockSpec((TOK_TILE, 1), lambda i: (i, 0)),
            pl.BlockSpec((NUM_CODE, C), lambda i: (0, 0)),
            pl.BlockSpec((1, NUM_CODE), lambda i: (0, 0)),
        ],
        out_specs=pl.BlockSpec((TOK_TILE, 1), lambda i: (i, 0)),
        out_shape=jax.ShapeDtypeStruct((N, 1), jnp.int32),
        compiler_params=pltpu.CompilerParams(
            dimension_semantics=("parallel",)),
    )(xb, x2, codebook, c2)

    indices_flat = idx2d.reshape(N)
    cb_pad = jnp.pad(codebook, ((0, 0), (0, 128 - C)))
    codes = _make_sc_gather(NUM_CODE, 128, N)(cb_pad, indices_flat)[:, :C]

    indices = indices_flat.reshape(B, H, W)
    z_q = jnp.transpose(codes.reshape(B, H, W, C), (0, 3, 1, 2))
    return (z_q, z_q, indices)


# speedup vs baseline: 1.7071x; 1.0186x over previous
import functools

import jax
import jax.numpy as jnp
from jax import lax
from jax.experimental import pallas as pl
from jax.experimental.pallas import tpu as pltpu
from jax.experimental.pallas import tpu_sc as plsc

NUM_CODE = 8192
CODE_DIM = 32
TOK_TILE = 1024
CHUNK = 2048
NCHUNK = NUM_CODE // CHUNK


def _vq_idx_kernel(xb_ref, x2_ref, cb_ref, c2_ref, idx_ref):
    xb = xb_ref[...].astype(jnp.float32)
    x2 = x2_ref[...]

    best_v = jnp.full((TOK_TILE, 1), jnp.inf, jnp.bfloat16)
    best_i = jnp.zeros((TOK_TILE, 1), jnp.int32)
    for t in range(NCHUNK):
        c_t = cb_ref[t * CHUNK:(t + 1) * CHUNK, :]
        c2_t = c2_ref[:, t * CHUNK:(t + 1) * CHUNK]
        d = lax.dot_general(xb, c_t, (((1,), (1,)), ((), ())),
                            precision=lax.Precision.DEFAULT,
                            preferred_element_type=jnp.float32)
        v = (x2 - 2.0 * d) + c2_t
        m = jnp.min(v, axis=1, keepdims=True)
        iota = lax.broadcasted_iota(jnp.int32, v.shape, 1)
        ii = jnp.min(jnp.where(v == m, iota, NUM_CODE),
                     axis=1, keepdims=True) + t * CHUNK
        upd = m < best_v.astype(jnp.float32)
        best_i = jnp.where(upd, ii, best_i)
        best_v = jnp.where(upd, m.astype(jnp.bfloat16), best_v)

    idx_ref[...] = best_i


def _make_sc_gather(V, D, B):
    info = plsc.get_sparse_core_info()
    NW = info.num_cores * info.num_subcores
    b_per_w = B // NW
    mesh = plsc.VectorSubcoreMesh(core_axis_name="c", subcore_axis_name="s")

    @functools.partial(
        pl.kernel, mesh=mesh,
        out_type=jax.ShapeDtypeStruct((B, D), jnp.float32),
        scratch_types=[
            pltpu.VMEM((b_per_w,), jnp.int32),
            pltpu.VMEM((b_per_w, D), jnp.float32),
            pltpu.SemaphoreType.DMA,
        ],
    )
    def gather(table_hbm, idx_hbm, out_hbm, idx_v, rows_v, sem):
        wid = lax.axis_index("s") * info.num_cores + lax.axis_index("c")
        base = wid * b_per_w
        pltpu.sync_copy(idx_hbm.at[pl.ds(base, b_per_w)], idx_v)
        pltpu.async_copy(table_hbm.at[idx_v], rows_v, sem).wait()
        pltpu.sync_copy(rows_v, out_hbm.at[pl.ds(base, b_per_w)])

    return gather


@jax.jit
def kernel(z_e, codebook):
    B, C, H, W = z_e.shape
    N = B * H * W
    z = jnp.transpose(z_e, (0, 2, 3, 1))
    flat = z.reshape(-1, C)
    x2 = jnp.sum(flat ** 2, axis=1, keepdims=True)
    c2 = jnp.sum(codebook ** 2, axis=1)[None, :]
    xb = flat.astype(jnp.bfloat16)

    grid = (N // TOK_TILE,)
    idx2d = pl.pallas_call(
        _vq_idx_kernel,
        grid=grid,
        in_specs=[
            pl.BlockSpec((TOK_TILE, C), lambda i: (i, 0)),
            pl.BlockSpec((TOK_TILE, 1), lambda i: (i, 0)),
            pl.BlockSpec((NUM_CODE, C), lambda i: (0, 0)),
            pl.BlockSpec((1, NUM_CODE), lambda i: (0, 0)),
        ],
        out_specs=pl.BlockSpec((TOK_TILE, 1), lambda i: (i, 0)),
        out_shape=jax.ShapeDtypeStruct((N, 1), jnp.int32),
        compiler_params=pltpu.CompilerParams(
            dimension_semantics=("parallel",)),
    )(xb, x2, codebook, c2)

    indices_flat = idx2d.reshape(N)
    cb_pad = jnp.pad(codebook, ((0, 0), (0, 128 - C)))
    codes = _make_sc_gather(NUM_CODE, 128, N)(cb_pad, indices_flat)[:, :C]

    indices = indices_flat.reshape(B, H, W)
    z_q = jnp.transpose(codes.reshape(B, H, W, C), (0, 3, 1, 2))
    return (z_q, z_q, indices)
